# Initial kernel scaffold; baseline (speedup 1.0000x reference)
import jax
import jax.numpy as jnp
from jax.experimental import pallas as pl


def kernel(filter_feature, s_in_w, s_conv_w, s_conv_b, s_xproj_w, s_dt_w, s_dt_b, s_A_log, s_D, s_out_w, t_in_w, t_conv_w, t_conv_b, t_xproj_w, t_dt_w, t_dt_b, t_A_log, t_D, t_out_w):
    raise NotImplementedError("write your pallas kernel here")


# speedup vs baseline: 5.6938x; 5.6938x over previous
import functools

import jax
import jax.numpy as jnp
from jax.experimental import pallas as pl
from jax.experimental.pallas import tpu as pltpu

D_CONV = 4


def _mamba_body(x_ref, in_w_ref, cw_ref, cb_ref, xproj_ref, dt_w_ref, dt_b_ref,
                A_T_ref, Dp_ref, out_w_ref, o_ref, dA_ref, dBx_ref, H_ref,
                *, L, d_inner, r, n, Lc):
    bf = jnp.bfloat16

    def rp(t):
        u = pltpu.bitcast(t, jnp.uint32)
        u = (u + jnp.uint32(0x7FFF) + ((u >> 16) & jnp.uint32(1))) \
            & jnp.uint32(0xFFFF0000)
        return pltpu.bitcast(u, jnp.float32)

    x = x_ref[0]
    xz = jnp.dot(rp(x).astype(bf), rp(in_w_ref[...]).astype(bf),
                 preferred_element_type=jnp.float32)
    xh = xz[:, :d_inner]
    z = xz[:, d_inner:]

    cw = cw_ref[...]
    xhb = rp(xh)
    acc = xhb * cw[3:4, :]
    for k in (1, 2, 3):
        shifted = jnp.concatenate(
            [jnp.zeros((k, d_inner), jnp.float32), xhb[:L - k]], axis=0)
        acc = acc + shifted * cw[3 - k:4 - k, :]
    xc = acc + cb_ref[...]
    xc = xc * jax.nn.sigmoid(xc)

    dbc = jnp.dot(rp(xc).astype(bf), rp(xproj_ref[...]).astype(bf),
                  preferred_element_type=jnp.float32)
    dt = jnp.dot(rp(dbc[:, :r]).astype(bf), rp(dt_w_ref[...]).astype(bf),
                 preferred_element_type=jnp.float32) + dt_b_ref[...]
    dt = jax.nn.softplus(dt)
    Bm = dbc[:, r:r + n]
    Cm = dbc[:, r + n:r + 2 * n]

    A_T = -jnp.exp(A_T_ref[...])
    dtxc = dt * xc

    h = jnp.zeros((n, d_inner), jnp.float32)
    ys = []
    for c in range(L // Lc):
        sl = slice(c * Lc, (c + 1) * Lc)
        dA_ref[...] = jnp.exp(dt[sl][:, None, :] * A_T[None, :, :])
        dBx_ref[...] = dtxc[sl][:, None, :] * Bm[sl][:, :, None]

        def body(t, hh):
            hh = dA_ref[t] * hh + dBx_ref[t]
            H_ref[t] = hh
            return hh

        h = jax.lax.fori_loop(0, Lc, body, h)
        Hb = rp(H_ref[...])
        Cb = rp(Cm[sl])
        ys.append(jnp.sum(Hb * Cb[:, :, None], axis=1))

    y = jnp.concatenate(ys, axis=0) + xc * Dp_ref[...]
    g = y * (z * jax.nn.sigmoid(z))
    o_ref[0] = jnp.dot(rp(g).astype(bf), rp(out_w_ref[...]).astype(bf),
                       preferred_element_type=jnp.float32)


def _mamba_block(x, in_w, conv_w, conv_b, xproj_w, dt_w, dt_b, A_log, Dp, out_w,
                 *, Lc=64, interpret=False):
    b, L, d_model = x.shape
    d_inner, n = A_log.shape
    r = dt_w.shape[0]
    cw = jnp.transpose(conv_w[:, 0, :])
    A_T = jnp.transpose(A_log)
    body = functools.partial(_mamba_body, L=L, d_inner=d_inner, r=r, n=n, Lc=Lc)
    full = lambda i: (0, 0)
    return pl.pallas_call(
        body,
        grid=(b,),
        in_specs=[
            pl.BlockSpec((1, L, d_model), lambda i: (i, 0, 0)),
            pl.BlockSpec((d_model, 2 * d_inner), full),
            pl.BlockSpec((D_CONV, d_inner), full),
            pl.BlockSpec((1, d_inner), full),
            pl.BlockSpec((d_inner, r + 2 * n), full),
            pl.BlockSpec((r, d_inner), full),
            pl.BlockSpec((1, d_inner), full),
            pl.BlockSpec((n, d_inner), full),
            pl.BlockSpec((1, d_inner), full),
            pl.BlockSpec((d_inner, d_model), full),
        ],
        out_specs=pl.BlockSpec((1, L, d_model), lambda i: (i, 0, 0)),
        out_shape=jax.ShapeDtypeStruct((b, L, d_model), jnp.float32),
        scratch_shapes=[
            pltpu.VMEM((Lc, n, d_inner), jnp.float32),
            pltpu.VMEM((Lc, n, d_inner), jnp.float32),
            pltpu.VMEM((Lc, n, d_inner), jnp.float32),
        ],
        compiler_params=pltpu.CompilerParams(
            dimension_semantics=("parallel",),
            vmem_limit_bytes=50 * 1024 * 1024,
        ),
        name="mamba_block",
        interpret=interpret,
    )(x, in_w, cw, conv_b.reshape(1, -1), xproj_w, dt_w,
      dt_b.reshape(1, -1), A_T, Dp.reshape(1, -1), out_w)


def kernel(filter_feature,
           s_in_w, s_conv_w, s_conv_b, s_xproj_w, s_dt_w, s_dt_b, s_A_log, s_D, s_out_w,
           t_in_w, t_conv_w, t_conv_b, t_xproj_w, t_dt_w, t_dt_b, t_A_log, t_D, t_out_w):
    h = _mamba_block(filter_feature, s_in_w, s_conv_w, s_conv_b, s_xproj_w,
                     s_dt_w, s_dt_b, s_A_log, s_D, s_out_w)
    h = jnp.swapaxes(h, 1, 2)
    h = _mamba_block(h, t_in_w, t_conv_w, t_conv_b, t_xproj_w,
                     t_dt_w, t_dt_b, t_A_log, t_D, t_out_w)
    return jnp.swapaxes(h, 1, 2)
